# Initial kernel scaffold; baseline (speedup 1.0000x reference)
import jax
import jax.numpy as jnp
from jax.experimental import pallas as pl


def kernel(x):
    raise NotImplementedError("write your pallas kernel here")


# speedup vs baseline: 1.4056x; 1.4056x over previous
import jax
import jax.numpy as jnp
from jax.experimental import pallas as pl
from jax.experimental.pallas import tpu as pltpu

_EPS = 0.001
_C = 256
_GROUP = 8


def _stats_kernel(x_ref, gram_ref, sums_ref):
    j = pl.program_id(1)

    @pl.when(j == 0)
    def _():
        gram_ref[...] = jnp.zeros_like(gram_ref)
        sums_ref[...] = jnp.zeros_like(sums_ref)

    blk = x_ref[...]
    g = jax.lax.dot_general(
        blk, blk, (((0,), (0,)), ((), ())),
        preferred_element_type=jnp.float32)
    s = jnp.sum(blk, axis=0, keepdims=True)
    gram_ref[...] += g[None]
    sums_ref[...] += jnp.broadcast_to(s[None], sums_ref.shape)


def _factor_kernel(gram_ref, sums_ref, n_total, wt_ref, bias_ref, a_s, sp_s, w_s):
    c = _C
    nf = jnp.float32(n_total)
    g = gram_ref[0] + gram_ref[1]
    srow = sums_ref[0, 0:1, :] + sums_ref[1, 0:1, :]
    m = srow / nf
    outer = jax.lax.dot_general(
        m, m, (((0,), (0,)), ((), ())),
        preferred_element_type=jnp.float32)
    cov = (g - nf * outer) / (nf - 1.0)
    lane_c = jax.lax.broadcasted_iota(jnp.int32, (1, c), 1)
    row_c = jax.lax.broadcasted_iota(jnp.int32, (c, 1), 0)
    eye = (lane_c == row_c).astype(jnp.float32)
    a_s[...] = (1.0 - _EPS) * cov + _EPS * eye
    sp_s[...] = jnp.zeros_like(sp_s)

    lane8 = jax.lax.broadcasted_iota(jnp.int32, (_GROUP, c), 1)
    sub8 = jax.lax.broadcasted_iota(jnp.int32, (_GROUP, c), 0)
    lane_m_sub = lane8 - sub8
    sub_col = jax.lax.broadcasted_iota(jnp.int32, (_GROUP, 1), 0)

    for grp in range(c // _GROUP):
        j0 = grp * _GROUP
        slab = sp_s[j0:j0 + _GROUP, :]
        arows = a_s[j0:j0 + _GROUP, :]
        sel = (lane_m_sub == j0)
        us = []
        uws = []
        for t in range(_GROUP):
            jcur = j0 + t
            v = arows[t:t + 1, :] - slab[t:t + 1, 0:c]
            vj = jax.lax.slice(v, (0, jcur), (1, jcur + 1))
            r = jax.lax.rsqrt(vj)
            u = jnp.where(lane_c >= jcur, v, 0.0) * r
            ej = jnp.where(lane_c == jcur, 1.0, 0.0)
            w = (ej - slab[t:t + 1, c:2 * c]) * r
            uw = jnp.concatenate([u, w], axis=1)
            cvec = jnp.sum(jnp.where(sel, jnp.broadcast_to(u, (_GROUP, c)), 0.0),
                           axis=1, keepdims=True)
            cmask = jnp.where(sub_col > t, cvec, 0.0)
            slab = slab + cmask * uw
            us.append(u)
            uws.append(uw)
        ug = jnp.concatenate(us, axis=0)
        uwg = jnp.concatenate(uws, axis=0)
        w_s[j0:j0 + _GROUP, :] = uwg[:, c:2 * c]
        sp_s[...] += jax.lax.dot_general(
            ug, uwg, (((0,), (0,)), ((), ())),
            preferred_element_type=jnp.float32)

    wv = w_s[...]
    wt_ref[...] = wv.T
    bias = jax.lax.dot_general(
        m, wv, (((1,), (1,)), ((), ())),
        preferred_element_type=jnp.float32)
    bias_ref[...] = jnp.broadcast_to(bias, bias_ref.shape)


def _whiten_kernel(x_ref, wt_ref, bias_ref, o_ref):
    blk = x_ref[...]
    o_ref[...] = jax.lax.dot_general(
        blk, wt_ref[...], (((1,), (0,)), ((), ())),
        preferred_element_type=jnp.float32) - bias_ref[0:1, :]


def kernel(x):
    b, w, h, c = x.shape
    n = b * w * h
    x2 = x.reshape(n, c)
    bn = 2048
    nb = n // (2 * bn)

    gram, sums = pl.pallas_call(
        _stats_kernel,
        grid=(2, nb),
        in_specs=[pl.BlockSpec((bn, c), lambda i, j: (i * nb + j, 0))],
        out_specs=[
            pl.BlockSpec((1, c, c), lambda i, j: (i, 0, 0)),
            pl.BlockSpec((1, 8, c), lambda i, j: (i, 0, 0)),
        ],
        out_shape=[
            jax.ShapeDtypeStruct((2, c, c), jnp.float32),
            jax.ShapeDtypeStruct((2, 8, c), jnp.float32),
        ],
        compiler_params=pltpu.CompilerParams(
            dimension_semantics=("parallel", "arbitrary")),
        name="decor_stats",
    )(x2)

    wt, bias = pl.pallas_call(
        lambda g_ref, s_ref, wt_ref, b_ref, a_s, sp_s, w_s: _factor_kernel(
            g_ref, s_ref, n, wt_ref, b_ref, a_s, sp_s, w_s),
        out_shape=[
            jax.ShapeDtypeStruct((c, c), jnp.float32),
            jax.ShapeDtypeStruct((8, c), jnp.float32),
        ],
        scratch_shapes=[
            pltpu.VMEM((c, c), jnp.float32),
            pltpu.VMEM((c, 2 * c), jnp.float32),
            pltpu.VMEM((c, c), jnp.float32),
        ],
        name="decor_factor",
    )(gram, sums)

    out2 = pl.pallas_call(
        _whiten_kernel,
        grid=(2, nb),
        in_specs=[
            pl.BlockSpec((bn, c), lambda i, j: (i * nb + j, 0)),
            pl.BlockSpec((c, c), lambda i, j: (0, 0)),
            pl.BlockSpec((8, c), lambda i, j: (0, 0)),
        ],
        out_specs=pl.BlockSpec((bn, c), lambda i, j: (i * nb + j, 0)),
        out_shape=jax.ShapeDtypeStruct((n, c), jnp.float32),
        compiler_params=pltpu.CompilerParams(
            dimension_semantics=("parallel", "arbitrary")),
        name="decor_whiten",
    )(x2, wt, bias)

    return out2.reshape(b, w, h, c)


# speedup vs baseline: 1.6233x; 1.1549x over previous
import jax
import jax.numpy as jnp
from jax.experimental import pallas as pl
from jax.experimental.pallas import tpu as pltpu

_EPS = 0.001
_C = 256
_GROUP = 8


def _stats_kernel(x_ref, gram_ref, sums_ref):
    j = pl.program_id(1)

    @pl.when(j == 0)
    def _():
        gram_ref[...] = jnp.zeros_like(gram_ref)
        sums_ref[...] = jnp.zeros_like(sums_ref)

    blk = x_ref[...]
    g = jax.lax.dot_general(
        blk, blk, (((0,), (0,)), ((), ())),
        preferred_element_type=jnp.float32)
    s = jnp.sum(blk, axis=0, keepdims=True)
    gram_ref[...] += g[None]
    sums_ref[...] += jnp.broadcast_to(s[None], sums_ref.shape)


def _factor_kernel(gram_ref, sums_ref, n_total, wt_ref, bias_ref, a_s, sp_s, w_s):
    c = _C
    nf = jnp.float32(n_total)
    g = gram_ref[0] + gram_ref[1]
    srow = sums_ref[0, 0:1, :] + sums_ref[1, 0:1, :]
    m = srow / nf
    outer = jax.lax.dot_general(
        m, m, (((0,), (0,)), ((), ())),
        preferred_element_type=jnp.float32)
    cov = (g - nf * outer) / (nf - 1.0)
    lane_c = jax.lax.broadcasted_iota(jnp.int32, (1, c), 1)
    row_c = jax.lax.broadcasted_iota(jnp.int32, (c, 1), 0)
    eye = (lane_c == row_c).astype(jnp.float32)
    a_s[...] = (1.0 - _EPS) * cov + _EPS * eye
    sp_s[...] = jnp.zeros_like(sp_s)

    lane8 = jax.lax.broadcasted_iota(jnp.int32, (_GROUP, c), 1)
    sub8 = jax.lax.broadcasted_iota(jnp.int32, (_GROUP, c), 0)
    lane_m_sub = lane8 - sub8
    sub_col = jax.lax.broadcasted_iota(jnp.int32, (_GROUP, 1), 0)

    for grp in range(c // _GROUP):
        j0 = grp * _GROUP
        slab = sp_s[j0:j0 + _GROUP, :]
        arows = a_s[j0:j0 + _GROUP, :]
        sel = (lane_m_sub == j0)
        us = []
        uws = []
        for t in range(_GROUP):
            jcur = j0 + t
            v = arows[t:t + 1, :] - slab[t:t + 1, 0:c]
            vj = jax.lax.slice(v, (0, jcur), (1, jcur + 1))
            r = jax.lax.rsqrt(vj)
            u = jnp.where(lane_c >= jcur, v, 0.0) * r
            ej = jnp.where(lane_c == jcur, 1.0, 0.0)
            w = (ej - slab[t:t + 1, c:2 * c]) * r
            uw = jnp.concatenate([u, w], axis=1)
            cvec = jnp.sum(jnp.where(sel, jnp.broadcast_to(u, (_GROUP, c)), 0.0),
                           axis=1, keepdims=True)
            cmask = jnp.where(sub_col > t, cvec, 0.0)
            slab = slab + cmask * uw
            us.append(u)
            uws.append(uw)
        ug = jnp.concatenate(us, axis=0)
        uwg = jnp.concatenate(uws, axis=0)
        w_s[j0:j0 + _GROUP, :] = uwg[:, c:2 * c]
        sp_s[...] += jax.lax.dot_general(
            ug, uwg, (((0,), (0,)), ((), ())),
            preferred_element_type=jnp.float32)

    wv = w_s[...]
    wt_ref[...] = wv.T
    bias = jax.lax.dot_general(
        m, wv, (((1,), (1,)), ((), ())),
        preferred_element_type=jnp.float32)
    bias_ref[...] = jnp.broadcast_to(bias, bias_ref.shape)


def _whiten_kernel(x_ref, wt_ref, bias_ref, o_ref):
    blk = x_ref[...]
    o_ref[...] = jax.lax.dot_general(
        blk, wt_ref[...], (((1,), (0,)), ((), ())),
        preferred_element_type=jnp.float32) - bias_ref[0:1, :]


def kernel(x):
    b, w, h, c = x.shape
    n = b * w * h
    x2 = x.reshape(n, c)
    bn = 4096
    nb = n // (2 * bn)

    gram, sums = pl.pallas_call(
        _stats_kernel,
        grid=(2, nb),
        in_specs=[pl.BlockSpec((bn, c), lambda i, j: (i * nb + j, 0))],
        out_specs=[
            pl.BlockSpec((1, c, c), lambda i, j: (i, 0, 0)),
            pl.BlockSpec((1, 8, c), lambda i, j: (i, 0, 0)),
        ],
        out_shape=[
            jax.ShapeDtypeStruct((2, c, c), jnp.float32),
            jax.ShapeDtypeStruct((2, 8, c), jnp.float32),
        ],
        compiler_params=pltpu.CompilerParams(
            dimension_semantics=("parallel", "arbitrary"),
            vmem_limit_bytes=50 * 1024 * 1024),
        name="decor_stats",
    )(x2)

    wt, bias = pl.pallas_call(
        lambda g_ref, s_ref, wt_ref, b_ref, a_s, sp_s, w_s: _factor_kernel(
            g_ref, s_ref, n, wt_ref, b_ref, a_s, sp_s, w_s),
        out_shape=[
            jax.ShapeDtypeStruct((c, c), jnp.float32),
            jax.ShapeDtypeStruct((8, c), jnp.float32),
        ],
        scratch_shapes=[
            pltpu.VMEM((c, c), jnp.float32),
            pltpu.VMEM((c, 2 * c), jnp.float32),
            pltpu.VMEM((c, c), jnp.float32),
        ],
        name="decor_factor",
    )(gram, sums)

    out2 = pl.pallas_call(
        _whiten_kernel,
        grid=(2, nb),
        in_specs=[
            pl.BlockSpec((bn, c), lambda i, j: (i * nb + j, 0)),
            pl.BlockSpec((c, c), lambda i, j: (0, 0)),
            pl.BlockSpec((8, c), lambda i, j: (0, 0)),
        ],
        out_specs=pl.BlockSpec((bn, c), lambda i, j: (i * nb + j, 0)),
        out_shape=jax.ShapeDtypeStruct((n, c), jnp.float32),
        compiler_params=pltpu.CompilerParams(
            dimension_semantics=("parallel", "arbitrary"),
            vmem_limit_bytes=50 * 1024 * 1024),
        name="decor_whiten",
    )(x2, wt, bias)

    return out2.reshape(b, w, h, c)


# speedup vs baseline: 1.7170x; 1.0577x over previous
import jax
import jax.numpy as jnp
from jax.experimental import pallas as pl
from jax.experimental.pallas import tpu as pltpu

_EPS = 0.001
_C = 256
_GROUP = 8


def _stats_kernel(x_ref, gram_ref, sums_ref):
    j = pl.program_id(1)

    @pl.when(j == 0)
    def _():
        gram_ref[...] = jnp.zeros_like(gram_ref)
        sums_ref[...] = jnp.zeros_like(sums_ref)

    blk = x_ref[...]
    g = jax.lax.dot_general(
        blk, blk, (((0,), (0,)), ((), ())),
        preferred_element_type=jnp.float32)
    s = jnp.sum(blk, axis=0, keepdims=True)
    gram_ref[...] += g[None]
    sums_ref[...] += jnp.broadcast_to(s[None], sums_ref.shape)


def _factor_kernel(gram_ref, sums_ref, n_total, wt_ref, bias_ref, a_s, sp_s, w_s):
    c = _C
    nf = jnp.float32(n_total)
    g = gram_ref[0] + gram_ref[1]
    srow = sums_ref[0, 0:1, :] + sums_ref[1, 0:1, :]
    m = srow / nf
    outer = jax.lax.dot_general(
        m, m, (((0,), (0,)), ((), ())),
        preferred_element_type=jnp.float32)
    cov = (g - nf * outer) / (nf - 1.0)
    lane_c = jax.lax.broadcasted_iota(jnp.int32, (1, c), 1)
    row_c = jax.lax.broadcasted_iota(jnp.int32, (c, 1), 0)
    eye = (lane_c == row_c).astype(jnp.float32)
    a_s[...] = (1.0 - _EPS) * cov + _EPS * eye
    sp_s[...] = jnp.zeros_like(sp_s)

    lane8 = jax.lax.broadcasted_iota(jnp.int32, (_GROUP, c), 1)
    sub8 = jax.lax.broadcasted_iota(jnp.int32, (_GROUP, c), 0)
    lane_m_sub = lane8 - sub8
    sub_col = jax.lax.broadcasted_iota(jnp.int32, (_GROUP, 1), 0)

    for grp in range(c // _GROUP):
        j0 = grp * _GROUP
        slab = sp_s[j0:j0 + _GROUP, :]
        arows = a_s[j0:j0 + _GROUP, :]
        sel = (lane_m_sub == j0)
        us = []
        uws = []
        for t in range(_GROUP):
            jcur = j0 + t
            v = arows[t:t + 1, :] - slab[t:t + 1, 0:c]
            vj = jax.lax.slice(v, (0, jcur), (1, jcur + 1))
            r = jax.lax.rsqrt(vj)
            u = jnp.where(lane_c >= jcur, v, 0.0) * r
            ej = jnp.where(lane_c == jcur, 1.0, 0.0)
            w = (ej - slab[t:t + 1, c:2 * c]) * r
            uw = jnp.concatenate([u, w], axis=1)
            cvec = jnp.sum(jnp.where(sel, jnp.broadcast_to(u, (_GROUP, c)), 0.0),
                           axis=1, keepdims=True)
            cmask = jnp.where(sub_col > t, cvec, 0.0)
            slab = slab + cmask * uw
            us.append(u)
            uws.append(uw)
        ug = jnp.concatenate(us, axis=0)
        uwg = jnp.concatenate(uws, axis=0)
        w_s[j0:j0 + _GROUP, :] = uwg[:, c:2 * c]
        sp_s[...] += jax.lax.dot_general(
            ug, uwg, (((0,), (0,)), ((), ())),
            preferred_element_type=jnp.float32)

    wv = w_s[...]
    wt_ref[...] = wv.T
    bias = jax.lax.dot_general(
        m, wv, (((1,), (1,)), ((), ())),
        preferred_element_type=jnp.float32)
    bias_ref[...] = jnp.broadcast_to(bias, bias_ref.shape)


def _whiten_kernel(x_ref, wt_ref, bias_ref, o_ref):
    blk = x_ref[...]
    o_ref[...] = jax.lax.dot_general(
        blk, wt_ref[...], (((1,), (0,)), ((), ())),
        preferred_element_type=jnp.float32) - bias_ref[0:1, :]


def kernel(x):
    b, w, h, c = x.shape
    n = b * w * h
    x2 = x.reshape(n, c)
    bn = 8192
    nb = n // (2 * bn)

    gram, sums = pl.pallas_call(
        _stats_kernel,
        grid=(2, nb),
        in_specs=[pl.BlockSpec((bn, c), lambda i, j: (i * nb + j, 0))],
        out_specs=[
            pl.BlockSpec((1, c, c), lambda i, j: (i, 0, 0)),
            pl.BlockSpec((1, 8, c), lambda i, j: (i, 0, 0)),
        ],
        out_shape=[
            jax.ShapeDtypeStruct((2, c, c), jnp.float32),
            jax.ShapeDtypeStruct((2, 8, c), jnp.float32),
        ],
        compiler_params=pltpu.CompilerParams(
            dimension_semantics=("parallel", "arbitrary"),
            vmem_limit_bytes=50 * 1024 * 1024),
        name="decor_stats",
    )(x2)

    wt, bias = pl.pallas_call(
        lambda g_ref, s_ref, wt_ref, b_ref, a_s, sp_s, w_s: _factor_kernel(
            g_ref, s_ref, n, wt_ref, b_ref, a_s, sp_s, w_s),
        out_shape=[
            jax.ShapeDtypeStruct((c, c), jnp.float32),
            jax.ShapeDtypeStruct((8, c), jnp.float32),
        ],
        scratch_shapes=[
            pltpu.VMEM((c, c), jnp.float32),
            pltpu.VMEM((c, 2 * c), jnp.float32),
            pltpu.VMEM((c, c), jnp.float32),
        ],
        name="decor_factor",
    )(gram, sums)

    out2 = pl.pallas_call(
        _whiten_kernel,
        grid=(2, nb),
        in_specs=[
            pl.BlockSpec((bn, c), lambda i, j: (i * nb + j, 0)),
            pl.BlockSpec((c, c), lambda i, j: (0, 0)),
            pl.BlockSpec((8, c), lambda i, j: (0, 0)),
        ],
        out_specs=pl.BlockSpec((bn, c), lambda i, j: (i * nb + j, 0)),
        out_shape=jax.ShapeDtypeStruct((n, c), jnp.float32),
        compiler_params=pltpu.CompilerParams(
            dimension_semantics=("parallel", "arbitrary"),
            vmem_limit_bytes=50 * 1024 * 1024),
        name="decor_whiten",
    )(x2, wt, bias)

    return out2.reshape(b, w, h, c)


# speedup vs baseline: 1.8765x; 1.0929x over previous
import jax
import jax.numpy as jnp
from jax.experimental import pallas as pl
from jax.experimental.pallas import tpu as pltpu

_EPS = 0.001
_C = 256
_GROUP = 16


def _stats_kernel(x_ref, gram_ref, sums_ref):
    j = pl.program_id(1)

    @pl.when(j == 0)
    def _():
        gram_ref[...] = jnp.zeros_like(gram_ref)
        sums_ref[...] = jnp.zeros_like(sums_ref)

    blk = x_ref[...]
    g = jax.lax.dot_general(
        blk, blk, (((0,), (0,)), ((), ())),
        preferred_element_type=jnp.float32)
    s = jnp.sum(blk, axis=0, keepdims=True)
    gram_ref[...] += g[None]
    sums_ref[...] += jnp.broadcast_to(s[None], sums_ref.shape)


def _factor_kernel(gram_ref, sums_ref, n_total, wt_ref, bias_ref, a_s, uw_s):
    c = _C
    gsz = _GROUP
    nf = jnp.float32(n_total)
    g = gram_ref[0] + gram_ref[1]
    srow = sums_ref[0, 0:1, :] + sums_ref[1, 0:1, :]
    m = srow / nf
    outer = jax.lax.dot_general(
        m, m, (((0,), (0,)), ((), ())),
        preferred_element_type=jnp.float32)
    cov = (g - nf * outer) / (nf - 1.0)
    lane_c = jax.lax.broadcasted_iota(jnp.int32, (1, c), 1)
    row_c = jax.lax.broadcasted_iota(jnp.int32, (c, 1), 0)
    eye = (lane_c == row_c).astype(jnp.float32)
    a_s[...] = (1.0 - _EPS) * cov + _EPS * eye
    uw_s[...] = jnp.zeros_like(uw_s)

    subg = jax.lax.broadcasted_iota(jnp.int32, (gsz, 1), 0)
    lane_g = jax.lax.broadcasted_iota(jnp.int32, (1, gsz), 1)
    eye_g = (lane_g == subg).astype(jnp.float32)

    for grp in range(c // gsz):
        j0 = grp * gsz
        arows = a_s[j0:j0 + gsz, :]
        eg = (lane_c == subg + j0).astype(jnp.float32)
        if grp == 0:
            slab_s, slab_p = arows, eg
        else:
            ucols = uw_s[:, j0:j0 + gsz]
            corr = jax.lax.dot_general(
                ucols, uw_s[...], (((0,), (0,)), ((), ())),
                preferred_element_type=jnp.float32)
            slab_s = arows - corr[:, 0:c]
            slab_p = eg - corr[:, c:2 * c]
        dblk = jax.lax.slice(slab_s, (0, j0), (gsz, j0 + gsz))
        mmat = jnp.concatenate([dblk, eye_g], axis=1)
        rs = []
        for t in range(gsz):
            dt = jax.lax.slice(mmat, (t, t), (t + 1, t + 1))
            rsq = jax.lax.rsqrt(dt)
            rc = rsq * rsq
            colt = jax.lax.slice(mmat, (0, t), (gsz, t + 1))
            prow = jax.lax.slice(mmat, (t, 0), (t + 1, 2 * gsz))
            coef = jnp.where(subg > t, colt, 0.0) * rc
            mmat = mmat - coef * prow
            rs.append(rsq)
        rvec = jnp.concatenate(rs, axis=0)
        linv = rvec * mmat[:, gsz:2 * gsz]
        ugwg = jax.lax.dot_general(
            linv, jnp.concatenate([slab_s, slab_p], axis=1),
            (((1,), (0,)), ((), ())),
            preferred_element_type=jnp.float32)
        ug = jnp.where(lane_c >= subg + j0, ugwg[:, 0:c], 0.0)
        wg = jnp.where(lane_c <= subg + j0, ugwg[:, c:2 * c], 0.0)
        uw_s[j0:j0 + gsz, :] = jnp.concatenate([ug, wg], axis=1)

    wv = uw_s[:, c:2 * c]
    wt_ref[...] = wv.T
    bias = jax.lax.dot_general(
        m, wv, (((1,), (1,)), ((), ())),
        preferred_element_type=jnp.float32)
    bias_ref[...] = jnp.broadcast_to(bias, bias_ref.shape)


def _whiten_kernel(x_ref, wt_ref, bias_ref, o_ref):
    blk = x_ref[...]
    o_ref[...] = jax.lax.dot_general(
        blk, wt_ref[...], (((1,), (0,)), ((), ())),
        preferred_element_type=jnp.float32) - bias_ref[0:1, :]


def kernel(x):
    b, w, h, c = x.shape
    n = b * w * h
    x2 = x.reshape(n, c)
    bn = 8192
    nb = n // (2 * bn)

    gram, sums = pl.pallas_call(
        _stats_kernel,
        grid=(2, nb),
        in_specs=[pl.BlockSpec((bn, c), lambda i, j: (i * nb + j, 0))],
        out_specs=[
            pl.BlockSpec((1, c, c), lambda i, j: (i, 0, 0)),
            pl.BlockSpec((1, 8, c), lambda i, j: (i, 0, 0)),
        ],
        out_shape=[
            jax.ShapeDtypeStruct((2, c, c), jnp.float32),
            jax.ShapeDtypeStruct((2, 8, c), jnp.float32),
        ],
        compiler_params=pltpu.CompilerParams(
            dimension_semantics=("parallel", "arbitrary"),
            vmem_limit_bytes=50 * 1024 * 1024),
        name="decor_stats",
    )(x2)

    wt, bias = pl.pallas_call(
        lambda g_ref, s_ref, wt_ref, b_ref, a_s, uw_s: _factor_kernel(
            g_ref, s_ref, n, wt_ref, b_ref, a_s, uw_s),
        out_shape=[
            jax.ShapeDtypeStruct((c, c), jnp.float32),
            jax.ShapeDtypeStruct((8, c), jnp.float32),
        ],
        scratch_shapes=[
            pltpu.VMEM((c, c), jnp.float32),
            pltpu.VMEM((c, 2 * c), jnp.float32),
        ],
        name="decor_factor",
    )(gram, sums)

    out2 = pl.pallas_call(
        _whiten_kernel,
        grid=(2, nb),
        in_specs=[
            pl.BlockSpec((bn, c), lambda i, j: (i * nb + j, 0)),
            pl.BlockSpec((c, c), lambda i, j: (0, 0)),
            pl.BlockSpec((8, c), lambda i, j: (0, 0)),
        ],
        out_specs=pl.BlockSpec((bn, c), lambda i, j: (i * nb + j, 0)),
        out_shape=jax.ShapeDtypeStruct((n, c), jnp.float32),
        compiler_params=pltpu.CompilerParams(
            dimension_semantics=("parallel", "arbitrary"),
            vmem_limit_bytes=50 * 1024 * 1024),
        name="decor_whiten",
    )(x2, wt, bias)

    return out2.reshape(b, w, h, c)


# speedup vs baseline: 1.9476x; 1.0379x over previous
import jax
import jax.numpy as jnp
from jax.experimental import pallas as pl
from jax.experimental.pallas import tpu as pltpu

_EPS = 0.001
_C = 256
_GROUP = 16


def _factor(gram, srow, n_total, wt_s, bias_s, a_s, uw_s):
    c = _C
    gsz = _GROUP
    nf = jnp.float32(n_total)
    m = srow / nf
    outer = jax.lax.dot_general(
        m, m, (((0,), (0,)), ((), ())),
        preferred_element_type=jnp.float32)
    cov = (gram - nf * outer) / (nf - 1.0)
    lane_c = jax.lax.broadcasted_iota(jnp.int32, (1, c), 1)
    row_c = jax.lax.broadcasted_iota(jnp.int32, (c, 1), 0)
    eye = (lane_c == row_c).astype(jnp.float32)
    a_s[...] = (1.0 - _EPS) * cov + _EPS * eye
    uw_s[...] = jnp.zeros_like(uw_s)

    subg = jax.lax.broadcasted_iota(jnp.int32, (gsz, 1), 0)
    lane_g = jax.lax.broadcasted_iota(jnp.int32, (1, gsz), 1)
    eye_g = (lane_g == subg).astype(jnp.float32)

    for grp in range(c // gsz):
        j0 = grp * gsz
        arows = a_s[j0:j0 + gsz, :]
        eg = (lane_c == subg + j0).astype(jnp.float32)
        if grp == 0:
            slab_s, slab_p = arows, eg
        else:
            ucols = uw_s[:, j0:j0 + gsz]
            corr = jax.lax.dot_general(
                ucols, uw_s[...], (((0,), (0,)), ((), ())),
                preferred_element_type=jnp.float32)
            slab_s = arows - corr[:, 0:c]
            slab_p = eg - corr[:, c:2 * c]
        dblk = jax.lax.slice(slab_s, (0, j0), (gsz, j0 + gsz))
        mmat = jnp.concatenate([dblk, eye_g], axis=1)
        rs = []
        for t in range(gsz):
            dt = jax.lax.slice(mmat, (t, t), (t + 1, t + 1))
            rsq = jax.lax.rsqrt(dt)
            rc = rsq * rsq
            colt = jax.lax.slice(mmat, (0, t), (gsz, t + 1))
            prow = jax.lax.slice(mmat, (t, 0), (t + 1, 2 * gsz))
            coef = jnp.where(subg > t, colt, 0.0) * rc
            mmat = mmat - coef * prow
            rs.append(rsq)
        rvec = jnp.concatenate(rs, axis=0)
        linv = rvec * mmat[:, gsz:2 * gsz]
        ugwg = jax.lax.dot_general(
            linv, jnp.concatenate([slab_s, slab_p], axis=1),
            (((1,), (0,)), ((), ())),
            preferred_element_type=jnp.float32)
        ug = jnp.where(lane_c >= subg + j0, ugwg[:, 0:c], 0.0)
        wg = jnp.where(lane_c <= subg + j0, ugwg[:, c:2 * c], 0.0)
        uw_s[j0:j0 + gsz, :] = jnp.concatenate([ug, wg], axis=1)

    wv = uw_s[:, c:2 * c]
    wt_s[...] = wv.T
    bias_s[...] = jax.lax.dot_general(
        m, wv, (((1,), (1,)), ((), ())),
        preferred_element_type=jnp.float32)


def _fused_kernel(nb, n_total, x_ref, o_ref,
                  gram_s, sum_s, a_s, uw_s, wt_s, bias_s):
    j = pl.program_id(0)

    @pl.when(j == 0)
    def _():
        gram_s[...] = jnp.zeros_like(gram_s)
        sum_s[...] = jnp.zeros_like(sum_s)

    @pl.when(j < nb)
    def _():
        blk = x_ref[...]
        gram_s[...] += jax.lax.dot_general(
            blk, blk, (((0,), (0,)), ((), ())),
            preferred_element_type=jnp.float32)
        sum_s[...] += jnp.sum(blk, axis=0, keepdims=True)

    @pl.when(j == nb)
    def _():
        _factor(gram_s[...], sum_s[...], n_total, wt_s, bias_s, a_s, uw_s)

    @pl.when(j > nb)
    def _():
        o_ref[...] = jax.lax.dot_general(
            x_ref[...], wt_s[...], (((1,), (0,)), ((), ())),
            preferred_element_type=jnp.float32) - bias_s[...]


def kernel(x):
    b, w, h, c = x.shape
    n = b * w * h
    x2 = x.reshape(n, c)
    bn = 8192
    nb = n // bn

    def x_map(j):
        return (jnp.where(j < nb, j, jnp.maximum(j - (nb + 1), 0)), 0)

    def o_map(j):
        return (jnp.maximum(j - (nb + 1), 0), 0)

    out2 = pl.pallas_call(
        lambda x_ref, o_ref, *scr: _fused_kernel(nb, n, x_ref, o_ref, *scr),
        grid=(2 * nb + 1,),
        in_specs=[pl.BlockSpec((bn, c), x_map)],
        out_specs=pl.BlockSpec((bn, c), o_map),
        out_shape=jax.ShapeDtypeStruct((n, c), jnp.float32),
        scratch_shapes=[
            pltpu.VMEM((c, c), jnp.float32),
            pltpu.VMEM((1, c), jnp.float32),
            pltpu.VMEM((c, c), jnp.float32),
            pltpu.VMEM((c, 2 * c), jnp.float32),
            pltpu.VMEM((c, c), jnp.float32),
            pltpu.VMEM((1, c), jnp.float32),
        ],
        compiler_params=pltpu.CompilerParams(
            dimension_semantics=("arbitrary",),
            vmem_limit_bytes=50 * 1024 * 1024),
        name="decor_fused",
    )(x2)

    return out2.reshape(b, w, h, c)


# speedup vs baseline: 2.0255x; 1.0400x over previous
import jax
import jax.numpy as jnp
from jax.experimental import pallas as pl
from jax.experimental.pallas import tpu as pltpu

_EPS = 0.001
_C = 256
_GROUP = 128


def _factor(gram, srow, n_total, wt_s, bias_s, a_s, uw_s):
    c = _C
    gsz = _GROUP
    nf = jnp.float32(n_total)
    m = srow / nf
    outer = jax.lax.dot_general(
        m, m, (((0,), (0,)), ((), ())),
        preferred_element_type=jnp.float32)
    cov = (gram - nf * outer) / (nf - 1.0)
    lane_c = jax.lax.broadcasted_iota(jnp.int32, (1, c), 1)
    row_c = jax.lax.broadcasted_iota(jnp.int32, (c, 1), 0)
    eye = (lane_c == row_c).astype(jnp.float32)
    a_s[...] = (1.0 - _EPS) * cov + _EPS * eye
    uw_s[...] = jnp.zeros_like(uw_s)

    subg = jax.lax.broadcasted_iota(jnp.int32, (gsz, 1), 0)
    lane_g = jax.lax.broadcasted_iota(jnp.int32, (1, gsz), 1)
    eye_g = (lane_g == subg).astype(jnp.float32)

    for grp in range(c // gsz):
        j0 = grp * gsz
        arows = a_s[j0:j0 + gsz, :]
        eg = (lane_c == subg + j0).astype(jnp.float32)
        if grp == 0:
            slab_s, slab_p = arows, eg
        else:
            ucols = uw_s[:, j0:j0 + gsz]
            corr = jax.lax.dot_general(
                ucols, uw_s[...], (((0,), (0,)), ((), ())),
                preferred_element_type=jnp.float32)
            slab_s = arows - corr[:, 0:c]
            slab_p = eg - corr[:, c:2 * c]
        dblk = jax.lax.slice(slab_s, (0, j0), (gsz, j0 + gsz))
        mmat = jnp.concatenate([dblk, eye_g], axis=1)
        rs = []
        for t in range(gsz):
            colt = jax.lax.slice(mmat, (0, t), (gsz, t + 1))
            dt = jax.lax.slice(colt, (t, 0), (t + 1, 1))
            rsq = jax.lax.rsqrt(dt)
            rc = rsq * rsq
            prow = jax.lax.slice(mmat, (t, 0), (t + 1, 2 * gsz))
            coef = jnp.where(subg > t, colt, 0.0) * rc
            mmat = mmat - coef * prow
            rs.append(rsq)
        rvec = jnp.concatenate(rs, axis=0)
        linv = rvec * mmat[:, gsz:2 * gsz]
        ugwg = jax.lax.dot_general(
            linv, jnp.concatenate([slab_s, slab_p], axis=1),
            (((1,), (0,)), ((), ())),
            preferred_element_type=jnp.float32)
        ug = jnp.where(lane_c >= subg + j0, ugwg[:, 0:c], 0.0)
        wg = jnp.where(lane_c <= subg + j0, ugwg[:, c:2 * c], 0.0)
        uw_s[j0:j0 + gsz, :] = jnp.concatenate([ug, wg], axis=1)

    wv = uw_s[:, c:2 * c]
    wt_s[...] = wv.T
    bias_s[...] = jax.lax.dot_general(
        m, wv, (((1,), (1,)), ((), ())),
        preferred_element_type=jnp.float32)


def _fused_kernel(nb, n_total, x_ref, o_ref,
                  gram_s, sum_s, a_s, uw_s, wt_s, bias_s):
    j = pl.program_id(0)

    @pl.when(j == 0)
    def _():
        gram_s[...] = jnp.zeros_like(gram_s)
        sum_s[...] = jnp.zeros_like(sum_s)

    @pl.when(j < nb)
    def _():
        blk = x_ref[...]
        gram_s[...] += jax.lax.dot_general(
            blk, blk, (((0,), (0,)), ((), ())),
            preferred_element_type=jnp.float32)
        sum_s[...] += jnp.sum(blk, axis=0, keepdims=True)

    @pl.when(j == nb)
    def _():
        _factor(gram_s[...], sum_s[...], n_total, wt_s, bias_s, a_s, uw_s)

    @pl.when(j > nb)
    def _():
        o_ref[...] = jax.lax.dot_general(
            x_ref[...], wt_s[...], (((1,), (0,)), ((), ())),
            preferred_element_type=jnp.float32) - bias_s[...]


def kernel(x):
    b, w, h, c = x.shape
    n = b * w * h
    x2 = x.reshape(n, c)
    bn = 8192
    nb = n // bn

    def x_map(j):
        return (jnp.where(j < nb, j, jnp.maximum(j - (nb + 1), 0)), 0)

    def o_map(j):
        return (jnp.maximum(j - (nb + 1), 0), 0)

    out2 = pl.pallas_call(
        lambda x_ref, o_ref, *scr: _fused_kernel(nb, n, x_ref, o_ref, *scr),
        grid=(2 * nb + 1,),
        in_specs=[pl.BlockSpec((bn, c), x_map)],
        out_specs=pl.BlockSpec((bn, c), o_map),
        out_shape=jax.ShapeDtypeStruct((n, c), jnp.float32),
        scratch_shapes=[
            pltpu.VMEM((c, c), jnp.float32),
            pltpu.VMEM((1, c), jnp.float32),
            pltpu.VMEM((c, c), jnp.float32),
            pltpu.VMEM((c, 2 * c), jnp.float32),
            pltpu.VMEM((c, c), jnp.float32),
            pltpu.VMEM((1, c), jnp.float32),
        ],
        compiler_params=pltpu.CompilerParams(
            dimension_semantics=("arbitrary",),
            vmem_limit_bytes=50 * 1024 * 1024),
        name="decor_fused",
    )(x2)

    return out2.reshape(b, w, h, c)
